# baseline (speedup 1.0000x reference)
import functools

import jax
import jax.numpy as jnp
import numpy as np
from jax import lax
from jax.experimental import pallas as pl
from jax.experimental.pallas import tpu as pltpu
from jax.experimental.pallas import tpu_sc as plsc

N = 10000
E = 320000
F = 128
FH = F // 2

NCORE = 2
NSUB = 16
C = 80
EP1 = E // NSUB
CH1 = EP1 // C
CH2 = CH1 // NCORE
NODE_T = 640
LAST_T = N - NODE_T * (NSUB - 1)
W1 = 16

_DEINT = np.empty(F, np.int32)
for _q in range(F):
    _hf, _qq = divmod(_q, FH)
    _g, _i = divmod(_qq, 32)
    _DEINT[_q] = _hf * FH + _g * 32 + 2 * (_i % 16) + (_i // 16)
_PMAT = np.zeros((F, F), np.float32)
_PMAT[np.arange(F), _DEINT] = 1.0


def _head_tc(x_ref, wt_ref, ap_ref, h1_ref, h2_ref, ab_ref):
    h = jnp.dot(x_ref[...], wt_ref[...], preferred_element_type=jnp.float32)
    h1_ref[...] = h[:, :FH].astype(jnp.bfloat16)
    h2_ref[...] = h[:, FH:].astype(jnp.bfloat16)
    ab_ref[...] = jnp.dot(h, ap_ref[...], preferred_element_type=jnp.float32)


def _add_tc(p_ref, pm_ref, o_ref):
    p = p_ref[...]
    res = jnp.concatenate([p[0, 0] + p[1, 0], p[0, 1] + p[1, 1]], axis=-1)
    o_ref[...] = jnp.dot(res, pm_ref[...], preferred_element_type=jnp.float32)


_mesh = plsc.VectorSubcoreMesh(core_axis_name="c", subcore_axis_name="s")


@functools.partial(
    pl.kernel,
    out_type=jax.ShapeDtypeStruct((NCORE, 2, N, FH), jnp.float32),
    mesh=_mesh,
    compiler_params=pltpu.CompilerParams(
        needs_layout_passes=False, use_tc_tiling_on_sc=False),
    scratch_types=[
        pltpu.VMEM((CH1, C), jnp.int32),
        pltpu.VMEM((CH1, C), jnp.int32),
        pltpu.VMEM((CH2, C), jnp.float32),
        pltpu.VMEM((4, C), jnp.float32),
        pltpu.VMEM((2 * N,), jnp.float32),
        pltpu.VMEM((2, C, FH), jnp.bfloat16),
        pltpu.VMEM((2, C, FH), jnp.float32),
        pltpu.VMEM((C,), jnp.float32),
        pltpu.VMEM((NODE_T,), jnp.float32),
        pltpu.VMEM_SHARED((N,), jnp.float32),
        pltpu.VMEM_SHARED((N, FH), jnp.float32),
        pltpu.SemaphoreType.DMA,
        pltpu.SemaphoreType.DMA,
        pltpu.SemaphoreType.DMA,
        pltpu.SemaphoreType.DMA,
        pltpu.SemaphoreType.DMA,
        pltpu.SemaphoreType.DMA,
        pltpu.SemaphoreType.DMA,
        pltpu.SemaphoreType.DMA,
        pltpu.SemaphoreType.DMA,
    ],
)
def _edge_sc(h1_hbm, h2_hbm, ab_hbm, edge_hbm, out_hbm,
             src2d, dst2d, exatt, exs, abl, rows, rowsf, zs, inv,
             s_sh, out_sh,
             sg0, sg1, ss0, ss1, ssp, sso0, sso1, sso2, sso3):
    cid = lax.axis_index("c")
    sid = lax.axis_index("s")
    z16 = jnp.zeros((16,), jnp.float32)
    base = cid * CH2
    obase = (1 - cid) * CH2
    r0 = sid * NODE_T
    nchunks = jnp.where(sid == NSUB - 1, LAST_T // C, NODE_T // C)
    sgs = (sg0, sg1)
    sss = (ss0, ss1)
    ssos = (sso0, sso1, sso2, sso3)

    @plsc.parallel_loop(0, C)
    def _zrow(r):
        for cb in range(FH // 16):
            rowsf[0, r, pl.ds(cb * 16, 16)] = z16

    for i in range(C // 16):
        zs[pl.ds(i * 16, 16)] = z16

    def _zinit(k, _):
        off = r0 + k * C
        pltpu.sync_copy(zs, s_sh.at[pl.ds(off, C)])
        pltpu.sync_copy(rowsf.at[0], out_sh.at[pl.ds(off, C), :])
        return 0
    lax.fori_loop(0, nchunks, _zinit, 0)

    pltpu.sync_copy(ab_hbm, abl)
    pltpu.sync_copy(edge_hbm.at[0, sid], src2d)
    pltpu.sync_copy(edge_hbm.at[1, sid], dst2d)

    for k in range(2):
        pltpu.async_copy(h1_hbm.at[dst2d.at[base + k]], rows.at[k], sgs[k])

    plsc.subcore_barrier()

    def _ex16(j, v):
        o = pl.ds(v * 16, 16)
        si = src2d[j, o]
        di = dst2d[j, o]
        ea = plsc.load_gather(abl, [si * 2])
        eb = plsc.load_gather(abl, [di * 2 + 1])
        e = ea + eb
        e = jnp.where(e >= 0.0, e, 0.2 * e)
        return jnp.exp(e)

    def _own_one(j0):
        j = base + j0
        for v in range(C // 16):
            exatt[j0, pl.ds(v * 16, 16)] = _ex16(j, v)
        pltpu.async_copy(exatt.at[j0], s_sh.at[src2d.at[j]], ssp, add=True)

    def _own_wait():
        pltpu.make_async_copy(exatt.at[0], s_sh.at[src2d.at[base]],
                              ssp).wait()

    def _p1a(j0, _):
        _own_one(j0)
        return 0
    lax.fori_loop(0, W1, _p1a, 0)

    def _p1b(j0, _):
        _own_one(j0)
        _own_wait()
        return 0
    lax.fori_loop(W1, CH2, _p1b, 0)

    def _p1d(k, _):
        _own_wait()
        return 0
    lax.fori_loop(0, W1, _p1d, 0)

    def _oth_cmp(j0, b):
        j = obase + j0
        for v in range(C // 16):
            exs[b, pl.ds(v * 16, 16)] = _ex16(j, v)

    def _oth_fire(j0, b):
        pltpu.async_copy(exs.at[b], s_sh.at[src2d.at[obase + j0]],
                         ssos[b], add=True)

    def _oth_wait(b):
        pltpu.make_async_copy(exs.at[b], s_sh.at[src2d.at[obase]],
                              ssos[b]).wait()

    for b in range(4):
        _oth_cmp(b, b)
        _oth_fire(b, b)

    def _p1o(i, _):
        for b in range(4):
            j0 = 4 * i + b
            _oth_wait(b)
            _oth_cmp(j0, b)
            _oth_fire(j0, b)
        return 0
    lax.fori_loop(1, (CH2 - 1) // 4, _p1o, 0)

    _oth_wait(0)
    _oth_cmp(CH2 - 1, 0)
    _oth_fire(CH2 - 1, 0)
    for b in range(4):
        _oth_wait(b)


    for f in range(2):
        table = h1_hbm if f == 0 else h2_hbm

        def _fire_g(j0c, slot, table=table):
            pltpu.async_copy(table.at[dst2d.at[base + j0c]], rows.at[slot],
                             sgs[slot])

        def _wait_g(slot, table=table):
            pltpu.make_async_copy(table.at[dst2d.at[base]], rows.at[slot],
                                  sgs[slot]).wait()

        def _fire_s(j0c, slot):
            pltpu.async_copy(rowsf.at[slot], out_sh.at[src2d.at[base + j0c]],
                             sss[slot], add=True)

        def _wait_s(slot):
            pltpu.make_async_copy(rowsf.at[slot], out_sh.at[src2d.at[base]],
                                  sss[slot]).wait()

        def _scale(j0, slot):
            rb = rows.at[slot]
            ro = rowsf.at[slot]

            @plsc.parallel_loop(0, C // 16)
            def _(gg):
                att16 = exatt[j0, pl.ds(gg * 16, 16)]
                for rr in range(16):
                    r = gg * 16 + rr
                    attv = jnp.take_along_axis(
                        att16, jnp.full((16,), rr, jnp.int32), axis=0)
                    for g in range(FH // 32):
                        vb = rb[r, pl.ds(g * 32, 32)]
                        av, bv = plsc.unpack(
                            vb, format=plsc.PackFormat.INTERLEAVED)
                        ro[r, pl.ds(g * 32, 16)] = av * attv
                        ro[r, pl.ds(g * 32 + 16, 16)] = bv * attv

        if f == 1:
            _fire_g(0, 0)
            _fire_g(1, 1)
        for b in range(2):
            _wait_g(b)
            _scale(jnp.int32(b), b)
            _fire_s(b, b)
            _fire_g(b + 2, b)

        def _main(i, _):
            for b in range(2):
                j0 = 2 * i + b
                _wait_g(b)
                _wait_s(b)
                _scale(j0, b)
                _fire_s(j0, b)
                _fire_g(jnp.minimum(j0 + 2, CH2 - 1), b)
            return 0
        lax.fori_loop(1, (CH2 - 1) // 2, _main, 0)

        _wait_g(0)
        _wait_s(0)
        _scale(jnp.int32(CH2 - 1), 0)
        _fire_s(CH2 - 1, 0)
        _wait_g(1)
        _wait_s(1)
        _wait_s(0)

        plsc.subcore_barrier()

        if f == 0:
            def _sload(k, _):
                off = k * C
                pltpu.sync_copy(s_sh.at[pl.ds(r0 + off, C)],
                                inv.at[pl.ds(off, C)])
                return 0
            lax.fori_loop(0, nchunks, _sload, 0)

            def _sinv(i, _):
                o = pl.ds(i * 16, 16)
                inv[o] = 1.0 / (inv[o] + 1e-16)
                return 0
            lax.fori_loop(0, nchunks * (C // 16), _sinv, 0)

        def _dump(k, _):
            off = r0 + k * C
            pltpu.sync_copy(out_sh.at[pl.ds(off, C), :], rowsf.at[0])

            @plsc.parallel_loop(0, C, unroll=2)
            def _norm(r, k=k):
                iv = plsc.load_gather(
                    inv, [jnp.full((16,), r, jnp.int32) + k * C])
                for cb in range(FH // 16):
                    o2 = pl.ds(cb * 16, 16)
                    rowsf[0, r, o2] = rowsf[0, r, o2] * iv
            pltpu.sync_copy(rowsf.at[0], out_hbm.at[cid, f, pl.ds(off, C), :])
            return 0
        lax.fori_loop(0, nchunks, _dump, 0)

        if f == 0:
            @plsc.parallel_loop(0, C)
            def _zrow2(r):
                for cb in range(FH // 16):
                    rowsf[0, r, pl.ds(cb * 16, 16)] = z16

            def _rez(k, _):
                pltpu.sync_copy(rowsf.at[0],
                                out_sh.at[pl.ds(r0 + k * C, C), :])
                return 0
            lax.fori_loop(0, nchunks, _rez, 0)
            plsc.subcore_barrier()


@jax.jit
def kernel(x, edge_index, W, a):
    wt = W.T
    ap = a.reshape(2, F).T
    h1, h2, ab = pl.pallas_call(
        _head_tc,
        out_shape=[
            jax.ShapeDtypeStruct((N, FH), jnp.bfloat16),
            jax.ShapeDtypeStruct((N, FH), jnp.bfloat16),
            jax.ShapeDtypeStruct((N, 2), jnp.float32),
        ],
    )(x, wt, ap)
    edge4 = edge_index.reshape(2, NSUB, CH1, C)
    parts = _edge_sc(h1, h2, ab.reshape(-1), edge4)
    out = pl.pallas_call(
        _add_tc,
        grid=(10,),
        in_specs=[
            pl.BlockSpec((NCORE, 2, N // 10, FH), lambda i: (0, 0, i, 0)),
            pl.BlockSpec((F, F), lambda i: (0, 0)),
        ],
        out_specs=pl.BlockSpec((N // 10, F), lambda i: (i, 0)),
        out_shape=jax.ShapeDtypeStruct((N, F), jnp.float32),
    )(parts, jnp.asarray(_PMAT))
    return out


# speedup vs baseline: 1.1244x; 1.1244x over previous
import functools

import jax
import jax.numpy as jnp
import numpy as np
from jax import lax
from jax.experimental import pallas as pl
from jax.experimental.pallas import tpu as pltpu
from jax.experimental.pallas import tpu_sc as plsc

N = 10000
E = 320000
F = 128
FH = F // 2

NCORE = 2
NSUB = 16
C = 80
EP1 = E // NSUB
CH1 = EP1 // C
CH2 = CH1 // NCORE
NODE_T = 640
LAST_T = N - NODE_T * (NSUB - 1)
W1 = 16

_DEINT = np.empty(F, np.int32)
for _q in range(F):
    _hf, _qq = divmod(_q, FH)
    _g, _i = divmod(_qq, 32)
    _DEINT[_q] = _hf * FH + _g * 32 + 2 * (_i % 16) + (_i // 16)
_PMAT = np.zeros((F, F), np.float32)
_PMAT[np.arange(F), _DEINT] = 1.0


def _head_tc(x_ref, wt_ref, ap_ref, h1_ref, h2_ref, ab_ref):
    h = jnp.dot(x_ref[...], wt_ref[...], preferred_element_type=jnp.float32)
    h1_ref[...] = h[:, :FH].astype(jnp.bfloat16)
    h2_ref[...] = h[:, FH:].astype(jnp.bfloat16)
    ab_ref[...] = jnp.dot(h, ap_ref[...], preferred_element_type=jnp.float32)


def _add_tc(p_ref, pm_ref, o_ref):
    p = p_ref[...]
    res = jnp.concatenate([p[0, 0] + p[1, 0], p[0, 1] + p[1, 1]], axis=-1)
    o_ref[...] = jnp.dot(res, pm_ref[...], preferred_element_type=jnp.float32)


_mesh = plsc.VectorSubcoreMesh(core_axis_name="c", subcore_axis_name="s")


@functools.partial(
    pl.kernel,
    out_type=jax.ShapeDtypeStruct((NCORE, 2, N, FH), jnp.float32),
    mesh=_mesh,
    compiler_params=pltpu.CompilerParams(
        needs_layout_passes=False, use_tc_tiling_on_sc=False),
    scratch_types=[
        pltpu.VMEM((CH1, C), jnp.int32),
        pltpu.VMEM((CH1, C), jnp.int32),
        pltpu.VMEM((CH2, C), jnp.float32),
        pltpu.VMEM((4, C), jnp.float32),
        pltpu.VMEM((2 * N,), jnp.float32),
        pltpu.VMEM((2, C, FH), jnp.bfloat16),
        pltpu.VMEM((2, C, FH), jnp.float32),
        pltpu.VMEM((C,), jnp.float32),
        pltpu.VMEM((NODE_T,), jnp.float32),
        pltpu.VMEM_SHARED((N,), jnp.float32),
        pltpu.VMEM_SHARED((N, FH), jnp.float32),
        pltpu.SemaphoreType.DMA,
        pltpu.SemaphoreType.DMA,
        pltpu.SemaphoreType.DMA,
        pltpu.SemaphoreType.DMA,
        pltpu.SemaphoreType.DMA,
        pltpu.SemaphoreType.DMA,
        pltpu.SemaphoreType.DMA,
        pltpu.SemaphoreType.DMA,
        pltpu.SemaphoreType.DMA,
    ],
)
def _edge_sc(h1_hbm, h2_hbm, ab_hbm, edge_hbm, out_hbm,
             src2d, dst2d, exatt, exs, abl, rows, rowsf, zs, inv,
             s_sh, out_sh,
             sg0, sg1, ss0, ss1, ssp, sso0, sso1, sso2, sso3):
    cid = lax.axis_index("c")
    sid = lax.axis_index("s")
    z16 = jnp.zeros((16,), jnp.float32)
    base = cid * CH2
    obase = (1 - cid) * CH2
    r0 = sid * NODE_T
    nchunks = jnp.where(sid == NSUB - 1, LAST_T // C, NODE_T // C)
    sgs = (sg0, sg1)
    sss = (ss0, ss1)
    ssos = (sso0, sso1, sso2, sso3)

    @plsc.parallel_loop(0, C)
    def _zrow(r):
        for cb in range(FH // 16):
            rowsf[0, r, pl.ds(cb * 16, 16)] = z16

    for i in range(C // 16):
        zs[pl.ds(i * 16, 16)] = z16

    def _zinit(k, _):
        off = r0 + k * C
        pltpu.sync_copy(zs, s_sh.at[pl.ds(off, C)])
        pltpu.sync_copy(rowsf.at[0], out_sh.at[pl.ds(off, C), :])
        return 0
    lax.fori_loop(0, nchunks, _zinit, 0)

    pltpu.sync_copy(ab_hbm, abl)
    pltpu.sync_copy(edge_hbm.at[0, sid], src2d)
    pltpu.sync_copy(edge_hbm.at[1, sid], dst2d)

    for k in range(2):
        pltpu.async_copy(h1_hbm.at[dst2d.at[base + k]], rows.at[k], sgs[k])

    plsc.subcore_barrier()

    def _ex16(j, v):
        o = pl.ds(v * 16, 16)
        si = src2d[j, o]
        di = dst2d[j, o]
        ea = plsc.load_gather(abl, [si * 2])
        eb = plsc.load_gather(abl, [di * 2 + 1])
        e = ea + eb
        e = jnp.where(e >= 0.0, e, 0.2 * e)
        return jnp.exp(e)

    def _own_one(j0):
        j = base + j0
        for v in range(C // 16):
            exatt[j0, pl.ds(v * 16, 16)] = _ex16(j, v)
        pltpu.async_copy(exatt.at[j0], s_sh.at[src2d.at[j]], ssp, add=True)

    def _own_wait():
        pltpu.make_async_copy(exatt.at[0], s_sh.at[src2d.at[base]],
                              ssp).wait()

    def _p1a(j0, _):
        _own_one(j0)
        return 0
    lax.fori_loop(0, W1, _p1a, 0)

    def _p1b(j0, _):
        _own_one(j0)
        _own_wait()
        return 0
    lax.fori_loop(W1, CH2, _p1b, 0)

    def _p1d(k, _):
        _own_wait()
        return 0
    lax.fori_loop(0, W1, _p1d, 0)

    def _oth_cmp(j0, b):
        j = obase + j0
        for v in range(C // 16):
            exs[b, pl.ds(v * 16, 16)] = _ex16(j, v)

    def _oth_fire(j0, b):
        pltpu.async_copy(exs.at[b], s_sh.at[src2d.at[obase + j0]],
                         ssos[b], add=True)

    def _oth_wait(b):
        pltpu.make_async_copy(exs.at[b], s_sh.at[src2d.at[obase]],
                              ssos[b]).wait()

    for b in range(4):
        _oth_cmp(b, b)
        _oth_fire(b, b)

    def _p1o(i, _):
        for b in range(4):
            j0 = 4 * i + b
            _oth_wait(b)
            _oth_cmp(j0, b)
            _oth_fire(j0, b)
        return 0
    lax.fori_loop(1, (CH2 - 1) // 4, _p1o, 0)

    _oth_wait(0)
    _oth_cmp(CH2 - 1, 0)
    _oth_fire(CH2 - 1, 0)
    for b in range(4):
        _oth_wait(b)


    for f in range(2):
        table = h1_hbm if f == 0 else h2_hbm

        def _fire_g(j0c, slot, table=table):
            pltpu.async_copy(table.at[dst2d.at[base + j0c]], rows.at[slot],
                             sgs[slot])

        def _wait_g(slot, table=table):
            pltpu.make_async_copy(table.at[dst2d.at[base]], rows.at[slot],
                                  sgs[slot]).wait()

        def _fire_s(j0c, slot):
            pltpu.async_copy(rowsf.at[slot], out_sh.at[src2d.at[base + j0c]],
                             sss[slot], add=True)

        def _wait_s(slot):
            pltpu.make_async_copy(rowsf.at[slot], out_sh.at[src2d.at[base]],
                                  sss[slot]).wait()

        def _scale(j0, slot):
            rb = rows.at[slot]
            ro = rowsf.at[slot]

            ar = exatt.at[j0]

            @plsc.parallel_loop(0, C, unroll=2)
            def _(r):
                attv = plsc.load_gather(ar, [jnp.full((16,), r, jnp.int32)])
                for g in range(FH // 32):
                    vb = rb[r, pl.ds(g * 32, 32)]
                    av, bv = plsc.unpack(
                        vb, format=plsc.PackFormat.INTERLEAVED)
                    ro[r, pl.ds(g * 32, 16)] = av * attv
                    ro[r, pl.ds(g * 32 + 16, 16)] = bv * attv

        if f == 1:
            _fire_g(0, 0)
            _fire_g(1, 1)
        for b in range(2):
            _wait_g(b)
            _scale(jnp.int32(b), b)
            _fire_s(b, b)
            _fire_g(b + 2, b)

        def _main(i, _):
            for b in range(2):
                j0 = 2 * i + b
                _wait_g(b)
                _wait_s(b)
                _scale(j0, b)
                _fire_s(j0, b)
                _fire_g(jnp.minimum(j0 + 2, CH2 - 1), b)
            return 0
        lax.fori_loop(1, (CH2 - 1) // 2, _main, 0)

        _wait_g(0)
        _wait_s(0)
        _scale(jnp.int32(CH2 - 1), 0)
        _fire_s(CH2 - 1, 0)
        _wait_g(1)
        _wait_s(1)
        _wait_s(0)

        plsc.subcore_barrier()

        if f == 0:
            def _sload(k, _):
                off = k * C
                pltpu.sync_copy(s_sh.at[pl.ds(r0 + off, C)],
                                inv.at[pl.ds(off, C)])
                return 0
            lax.fori_loop(0, nchunks, _sload, 0)

            def _sinv(i, _):
                o = pl.ds(i * 16, 16)
                inv[o] = 1.0 / (inv[o] + 1e-16)
                return 0
            lax.fori_loop(0, nchunks * (C // 16), _sinv, 0)

        def _dump(k, _):
            off = r0 + k * C
            pltpu.sync_copy(out_sh.at[pl.ds(off, C), :], rowsf.at[0])

            @plsc.parallel_loop(0, C, unroll=2)
            def _norm(r, k=k):
                iv = plsc.load_gather(
                    inv, [jnp.full((16,), r, jnp.int32) + k * C])
                for cb in range(FH // 16):
                    o2 = pl.ds(cb * 16, 16)
                    rowsf[0, r, o2] = rowsf[0, r, o2] * iv
            pltpu.sync_copy(rowsf.at[0], out_hbm.at[cid, f, pl.ds(off, C), :])
            return 0
        lax.fori_loop(0, nchunks, _dump, 0)

        if f == 0:
            @plsc.parallel_loop(0, C)
            def _zrow2(r):
                for cb in range(FH // 16):
                    rowsf[0, r, pl.ds(cb * 16, 16)] = z16

            def _rez(k, _):
                pltpu.sync_copy(rowsf.at[0],
                                out_sh.at[pl.ds(r0 + k * C, C), :])
                return 0
            lax.fori_loop(0, nchunks, _rez, 0)
            plsc.subcore_barrier()


@jax.jit
def kernel(x, edge_index, W, a):
    wt = W.T
    ap = a.reshape(2, F).T
    h1, h2, ab = pl.pallas_call(
        _head_tc,
        out_shape=[
            jax.ShapeDtypeStruct((N, FH), jnp.bfloat16),
            jax.ShapeDtypeStruct((N, FH), jnp.bfloat16),
            jax.ShapeDtypeStruct((N, 2), jnp.float32),
        ],
    )(x, wt, ap)
    edge4 = edge_index.reshape(2, NSUB, CH1, C)
    parts = _edge_sc(h1, h2, ab.reshape(-1), edge4)
    out = pl.pallas_call(
        _add_tc,
        grid=(10,),
        in_specs=[
            pl.BlockSpec((NCORE, 2, N // 10, FH), lambda i: (0, 0, i, 0)),
            pl.BlockSpec((F, F), lambda i: (0, 0)),
        ],
        out_specs=pl.BlockSpec((N // 10, F), lambda i: (i, 0)),
        out_shape=jax.ShapeDtypeStruct((N, F), jnp.float32),
    )(parts, jnp.asarray(_PMAT))
    return out
